# baseline (speedup 1.0000x reference)
import functools

import jax
import jax.numpy as jnp
from jax import lax
from jax.experimental import pallas as pl
from jax.experimental.pallas import tpu as pltpu
from jax.experimental.pallas import tpu_sc as plsc

_N = 2048
_P = 5
_NW = 32
_RPW = _N // _NW
_CR = 4
_NCHUNK = _RPW // _CR
_UNROLL = 4


def _sc_body(planes_hbm, tbl_hbm, out_hbm,
             a0, a1, a2, a3, a4, b0, b1, b2, b3, b4,
             oa, ob, tbl_v,
             isem_a, isem_b, osem_a, osem_b):
    bufs = ((a0, a1, a2, a3, a4), (b0, b1, b2, b3, b4))
    obufs = (oa, ob)
    isems = (isem_a, isem_b)
    osems = (osem_a, osem_b)
    w = lax.axis_index("s") * 2 + lax.axis_index("c")
    row0 = w * _RPW
    pltpu.sync_copy(tbl_hbm, tbl_v)
    tbl = tbl_v[...]
    dnums = lax.GatherDimensionNumbers(
        offset_dims=(), collapsed_slice_dims=(0,), start_index_map=(0,))

    def start_loads(c, par):
        r = row0 + c * _CR
        return [
            pltpu.async_copy(
                planes_hbm.at[pl.ds(k * _N + r, _CR), :],
                bufs[par][k], isems[par])
            for k in range(_P)
        ]

    pending_in = {0: start_loads(0, 0)}
    pending_out = {}
    for c in range(_NCHUNK):
        par = c % 2
        for h in pending_in.pop(c):
            h.wait()
        if c + 1 < _NCHUNK:
            pending_in[c + 1] = start_loads(c + 1, 1 - par)
        if c - 2 in pending_out:
            pending_out.pop(c - 2).wait()
        cur = bufs[par]
        obuf = obufs[par]
        for rr in range(_CR):
            def step(j, _, rr=rr, cur=cur, obuf=obuf):
                for u in range(_UNROLL):
                    cc = (j * _UNROLL + u) * 16
                    inv = lax.shift_right_logical(cur[0][rr, pl.ds(cc, 16)], 31)
                    for k in range(1, _P):
                        inv = inv + lax.shift_right_logical(
                            cur[k][rr, pl.ds(cc, 16)], 31)
                    obuf[rr, pl.ds(cc, 16)] = lax.gather(
                        tbl, inv[:, None], dnums, (1,),
                        mode=lax.GatherScatterMode.PROMISE_IN_BOUNDS)
                return 0

            lax.fori_loop(0, _N // 16 // _UNROLL, step, 0)
        r = row0 + c * _CR
        pending_out[c] = pltpu.async_copy(
            obuf, out_hbm.at[pl.ds(r, _CR), :], osems[par])
    for c in sorted(pending_out):
        pending_out.pop(c).wait()


@jax.jit
def kernel(x, paths, b):
    del x
    planes = jnp.moveaxis(paths, -1, 0).reshape(_P * _N, _N)
    tbl = jnp.concatenate([b[::-1], jnp.zeros((11,), jnp.float32)])
    ibuf = pltpu.VMEM((_CR, _N), jnp.int32)
    obuf = pltpu.VMEM((_CR, _N), jnp.float32)
    run = functools.partial(
        pl.kernel,
        mesh=plsc.VectorSubcoreMesh(core_axis_name="c", subcore_axis_name="s"),
        out_type=jax.ShapeDtypeStruct((_N, _N), jnp.float32),
        scratch_types=[
            ibuf, ibuf, ibuf, ibuf, ibuf,
            ibuf, ibuf, ibuf, ibuf, ibuf,
            obuf, obuf,
            pltpu.VMEM((16,), jnp.float32),
            pltpu.SemaphoreType.DMA,
            pltpu.SemaphoreType.DMA,
            pltpu.SemaphoreType.DMA,
            pltpu.SemaphoreType.DMA,
        ],
    )(_sc_body)
    return run(planes, tbl)


# speedup vs baseline: 2.4830x; 2.4830x over previous
import jax
import jax.numpy as jnp
from jax.experimental import pallas as pl
from jax.experimental.pallas import tpu as pltpu

_N = 2048
_P = 5
_BR = 256


def _body(tab_ref, p_ref, o_ref):
    inv = jax.lax.shift_right_logical(p_ref[0], 31)
    for k in range(1, _P):
        inv = inv + jax.lax.shift_right_logical(p_ref[k], 31)
    counts = _P - inv
    out = jnp.where(counts == 0, jnp.float32(0.0), tab_ref[0])
    out = jnp.where(counts == 2, tab_ref[1], out)
    out = jnp.where(counts == 3, tab_ref[2], out)
    out = jnp.where(counts == 4, tab_ref[3], out)
    out = jnp.where(counts == 5, tab_ref[4], out)
    o_ref[...] = out


@jax.jit
def kernel(x, paths, b):
    del x
    planes = jnp.moveaxis(paths, -1, 0)
    grid = (_N // _BR,)
    return pl.pallas_call(
        _body,
        grid=grid,
        in_specs=[
            pl.BlockSpec(memory_space=pltpu.SMEM),
            pl.BlockSpec((_P, _BR, _N), lambda i: (0, i, 0)),
        ],
        out_specs=pl.BlockSpec((_BR, _N), lambda i: (i, 0)),
        out_shape=jax.ShapeDtypeStruct((_N, _N), jnp.float32),
        compiler_params=pltpu.CompilerParams(
            dimension_semantics=("arbitrary",),
        ),
    )(b, planes)
